# baseline (speedup 1.0000x reference)
import functools

import jax
import jax.numpy as jnp
from jax import lax
from jax.experimental import pallas as pl
from jax.experimental.pallas import tpu as pltpu
from jax.experimental.pallas import tpu_sc as plsc

B = 4096
H = 200
HP = 256
HA = 128
HB = H - HA
D = 64
TV = 500000
L = 16
ND = D // L
NBUF = 2


def kernel(indices, table):
    info = plsc.get_sparse_core_info()
    nw = info.num_cores * info.num_subcores
    bpw = B // nw
    idxp = jnp.pad(indices.astype(jnp.int32), ((0, 0), (0, HP - H)))
    ih = (idxp >> 1).reshape(2 * B, 2 * D)
    pa = (idxp & 1).astype(jnp.float32).reshape(2 * B, 2 * D)

    mesh = plsc.VectorSubcoreMesh(core_axis_name="c", subcore_axis_name="s")

    @functools.partial(
        pl.kernel,
        out_type=jax.ShapeDtypeStruct((B // 2, 2 * D), jnp.float32),
        mesh=mesh,
        compiler_params=pltpu.CompilerParams(use_tc_tiling_on_sc=True),
        scratch_types=[
            pltpu.VMEM((2 * bpw, 2 * D), jnp.int32),
            pltpu.VMEM((2 * bpw, 2 * D), jnp.float32),
            pltpu.VMEM((NBUF, H, 2 * D), jnp.float32),
            pltpu.VMEM((bpw // 2, 2 * D), jnp.float32),
        ] + [pltpu.SemaphoreType.DMA] * NBUF,
    )
    def run(ih_hbm, pa_hbm, tab_hbm, out_hbm, ih_v, pa_v, rows_v, out_v, *sems):
        wid = lax.axis_index("s") * info.num_cores + lax.axis_index("c")
        base = wid * bpw
        pltpu.sync_copy(ih_hbm.at[pl.ds(base * 2, 2 * bpw)], ih_v)
        pltpu.sync_copy(pa_hbm.at[pl.ds(base * 2, 2 * bpw)], pa_v)

        rows = tuple(rows_v.at[k] for k in range(NBUF))

        def fire(b, k):
            pltpu.async_copy(
                tab_hbm.at[ih_v.at[2 * b]], rows[k].at[pl.ds(0, HA)], sems[k]
            )
            pltpu.async_copy(
                tab_hbm.at[ih_v.at[2 * b + 1, pl.ds(0, HB)]],
                rows[k].at[pl.ds(HA, HB)],
                sems[k],
            )

        def drain(k):
            pltpu.make_async_copy(tab_hbm.at[pl.ds(0, H)], rows[k], sems[k]).wait()

        def row_add(rref, r, pvec, j, acc):
            par = jnp.broadcast_to(pvec[j : j + 1], (L,))
            out = []
            for d in range(ND):
                lo = rref[r, pl.ds(L * d, L)]
                hi = rref[r, pl.ds(D + L * d, L)]
                out.append(acc[d] + (lo + par * (hi - lo)))
            return tuple(out)

        def accum(b, rref, orow, ocol):
            zeros = tuple(jnp.zeros((L,), jnp.float32) for _ in range(ND))

            def gbody(g, acc):
                in_hi = (g >= HA // L).astype(jnp.int32)
                prow = 2 * b + in_hi
                pcol = L * g - HA * in_hi
                pvec = pa_v[prow, pl.ds(pcol, L)]
                for j in range(L):
                    acc = row_add(rref, L * g + j, pvec, j, acc)
                return acc

            acc = lax.fori_loop(0, (H - L // 2) // L, gbody, zeros)
            pvec = pa_v[2 * b + 1, pl.ds(HB - L // 2, L)]
            for j in range(L // 2):
                acc = row_add(rref, (H // L) * L + j, pvec, j, acc)
            for d in range(ND):
                out_v[orow, pl.ds(ocol + L * d, L)] = acc[d]

        for k in range(NBUF - 1):
            fire(k, k)

        def body(g, carry):
            b0 = NBUF * g
            for k in range(NBUF):
                b = b0 + k

                @pl.when(b + NBUF - 1 < bpw)
                def _(b=b, k=k):
                    fire(b + NBUF - 1, (k + NBUF - 1) % NBUF)

                drain(k)
                accum(b, rows[k], g, D * k)
            return carry

        lax.fori_loop(0, bpw // NBUF, body, 0)

        pltpu.sync_copy(out_v, out_hbm.at[pl.ds(wid * (bpw // 2), bpw // 2)])

    t128 = table[: 2 * TV].reshape(TV, 2 * D)
    return run(ih, pa, t128).reshape(B, D)


# speedup vs baseline: 1.9376x; 1.9376x over previous
import functools

import jax
import jax.numpy as jnp
from jax import lax
from jax.experimental import pallas as pl
from jax.experimental.pallas import tpu as pltpu
from jax.experimental.pallas import tpu_sc as plsc

B = 4096
H = 200
HP = 256
HQ = 64
HR = H - 3 * HQ
D = 64
L = 16
ND = D // L
NBUF = 6


def kernel(indices, table):
    info = plsc.get_sparse_core_info()
    nw = info.num_cores * info.num_subcores
    bpw = B // nw
    idxp = jnp.pad(indices.astype(jnp.int32), ((0, 0), (0, HP - H)))
    idx4 = idxp.reshape(4 * B, HQ)

    mesh = plsc.VectorSubcoreMesh(core_axis_name="c", subcore_axis_name="s")

    @functools.partial(
        pl.kernel,
        out_type=jax.ShapeDtypeStruct((B, D), jnp.float32),
        mesh=mesh,
        compiler_params=pltpu.CompilerParams(use_tc_tiling_on_sc=False),
        scratch_types=[
            pltpu.VMEM((4 * bpw, HQ), jnp.int32),
            pltpu.VMEM((NBUF, H, D), jnp.float32),
            pltpu.VMEM((bpw, D), jnp.float32),
        ] + [pltpu.SemaphoreType.DMA] * NBUF,
    )
    def run(idx_hbm, tab_hbm, out_hbm, idx_v, rows_v, out_v, *sems):
        wid = lax.axis_index("s") * info.num_cores + lax.axis_index("c")
        base = wid * bpw
        pltpu.sync_copy(idx_hbm.at[pl.ds(base * 4, 4 * bpw)], idx_v)

        rows = tuple(rows_v.at[k] for k in range(NBUF))

        def fire(b, k):
            for q in range(3):
                pltpu.async_copy(
                    tab_hbm.at[idx_v.at[4 * b + q]],
                    rows[k].at[pl.ds(HQ * q, HQ)],
                    sems[k],
                )
            pltpu.async_copy(
                tab_hbm.at[idx_v.at[4 * b + 3, pl.ds(0, HR)]],
                rows[k].at[pl.ds(3 * HQ, HR)],
                sems[k],
            )

        def drain(k):
            pltpu.make_async_copy(tab_hbm.at[pl.ds(0, H)], rows[k], sems[k]).wait()

        def accum(b, rref):
            def rbody(g, acc):
                for j in range(8):
                    r = g * 8 + j
                    acc = tuple(
                        acc[d] + rref[r, pl.ds(L * d, L)] for d in range(ND)
                    )
                return acc

            acc = lax.fori_loop(
                0, H // 8, rbody,
                tuple(jnp.zeros((L,), jnp.float32) for _ in range(ND)),
            )
            for d in range(ND):
                out_v[b, pl.ds(L * d, L)] = acc[d]

        for k in range(NBUF - 1):
            fire(k, k)

        nfull = bpw // NBUF

        def body(g, carry):
            b0 = NBUF * g
            for k in range(NBUF):
                b = b0 + k

                @pl.when(b + NBUF - 1 < bpw)
                def _(b=b, k=k):
                    fire(b + NBUF - 1, (k + NBUF - 1) % NBUF)

                drain(k)
                accum(b, rows[k])
            return carry

        lax.fori_loop(0, nfull, body, 0)
        for k in range(bpw - NBUF * nfull):
            drain(k)
            accum(NBUF * nfull + k, rows[k])

        pltpu.sync_copy(out_v, out_hbm.at[pl.ds(base, bpw)])

    return run(idx4, table)
